# baseline (speedup 1.0000x reference)
import jax
import jax.numpy as jnp
from jax import lax
from jax.experimental import pallas as pl
from jax.experimental.pallas import tpu as pltpu

_VMEM_LIMIT = 64 * 1024 * 1024


def _xform_kernel(x_ref, w_ref, dis_ref, y_ref):
    xw = jnp.dot(x_ref[...], w_ref[...], preferred_element_type=jnp.float32)
    y_ref[...] = (dis_ref[...] * xw).astype(y_ref.dtype)


def _make_agg_kernel(nch, tm, ts, c):
    def _contrib(srcl_ref, dstl_ref, y_ref, jg, st):
        srcl = srcl_ref[jg, 0, :]
        dstl = dstl_ref[jg, 0, :]
        ys = y_ref[pl.ds(pl.multiple_of(st * ts, ts), ts), :]
        s_oht = (lax.broadcasted_iota(jnp.int32, (ts, c), 0)
                 == srcl[None, :]).astype(jnp.bfloat16)
        g = lax.dot_general(s_oht, ys, (((0,), (0,)), ((), ())),
                            preferred_element_type=jnp.float32)
        d_oh = (lax.broadcasted_iota(jnp.int32, (tm, c), 0)
                == dstl[None, :]).astype(jnp.bfloat16)
        return jnp.dot(d_oh, g.astype(jnp.bfloat16),
                       preferred_element_type=jnp.float32)

    def _agg_kernel(meta_ref, srcl_ref, dstl_ref, y_ref, dis_ref, b_ref,
                    o_ref, acc_ref):
        h = pl.program_id(0)
        j = pl.program_id(1)
        jg = h * nch + 2 * j
        dt = meta_ref[0, jg]

        @pl.when(meta_ref[2, jg] == 1)
        def _():
            acc_ref[...] = jnp.zeros_like(acc_ref)

        @pl.when(meta_ref[4, jg] == 1)
        def _():
            acc_ref[...] += (
                _contrib(srcl_ref, dstl_ref, y_ref, jg, meta_ref[1, jg])
                + _contrib(srcl_ref, dstl_ref, y_ref, jg + 1,
                           meta_ref[1, jg + 1]))

        @pl.when(meta_ref[3, jg + 1] == 1)
        def _():
            yd = y_ref[pl.ds(pl.multiple_of(dt * tm, tm), tm), :]
            dis = dis_ref[pl.ds(pl.multiple_of(dt * tm, tm), tm), :]
            o_ref[...] = (dis * (acc_ref[...] + yd.astype(jnp.float32))
                          + b_ref[...])

    return _agg_kernel


def kernel(x, edge_index, weight, bias):
    n, nfeat = x.shape
    nhid = weight.shape[1]
    e = edge_index.shape[1]
    i32 = jnp.int32

    tm = 512
    ts = 512
    c = 128
    nt = n // tm
    ns = n // ts
    nth = nt // 2
    npair = nt * ns
    nch = e // c + 1 + nth * ns + 3 * nth + 1
    nch += nch & 1
    nc = 2 * nch

    src = edge_index[0]
    dst = edge_index[1]

    deg = jnp.zeros((n,), jnp.float32).at[dst].add(1.0) + 1.0
    d_is = lax.rsqrt(deg)[:, None]

    sb = (n - 1).bit_length()
    db = (tm - 1).bit_length()
    dtile = dst // tm
    stile = src // ts
    key = (dtile * ns + stile).astype(i32)
    kv = jnp.sort((key << (sb + db)) | (src << db) | (dst % tm))
    skey = kv >> (sb + db)
    src_s = (kv >> db) & (n - 1)
    srcl_s = src_s % ts
    stile_s = src_s // ts
    dstl_s = kv & (tm - 1)

    cnt = jnp.zeros((npair,), i32).at[key].add(1)
    ch_p = (cnt + c - 1) // c
    tile_ch = ch_p.reshape(nt, ns).sum(1)
    tile_ch_m = jnp.maximum(tile_ch, 1)
    tile_ch_m = tile_ch_m + (tile_ch_m & 1)
    d_t = jnp.cumsum(tile_ch) - tile_ch
    g_ex = jnp.cumsum(tile_ch_m) - tile_ch_m
    half = (jnp.arange(nt, dtype=i32) >= nth).astype(i32)
    gtile = half * nch + g_ex - half * g_ex[nth]

    idx = jnp.arange(e, dtype=i32)
    chg = jnp.concatenate([jnp.ones((1,), i32),
                           (skey[1:] != skey[:-1]).astype(i32)])
    run_start = lax.cummax(idx * chg)
    r = idx - run_start
    c_ord = jnp.cumsum((r % c == 0).astype(i32)) - 1
    adj = gtile - d_t
    dtile_e = skey // ns
    oh16 = (dtile_e[:, None] == jnp.arange(nt, dtype=i32)[None, :])
    slot = c_ord + jnp.sum(oh16 * adj[None, :], axis=1)
    pos = slot * c + r % c

    pack = (stile_s << 20) | (srcl_s << 10) | dstl_s
    sentp = (1023 << 10) | 1023
    buf = jnp.full((nc * c,), sentp, i32).at[pos].add(pack - sentp)
    buf2 = buf.reshape(nc, c)
    srcl3 = ((buf2 >> 10) & 1023).reshape(nc, 1, c)
    dstl3 = (buf2 & 1023).reshape(nc, 1, c)
    head = buf2[:, 0]
    ch_stile = head >> 20
    ch_valid = (head != sentp).astype(i32)

    j = jnp.arange(nc, dtype=i32)
    ch_first = jnp.any(j[:, None] == gtile[None, :], axis=1).astype(i32)
    ch_last = jnp.any(j[:, None] == (gtile + tile_ch_m - 1)[None, :],
                      axis=1).astype(i32)
    ch_dtile = jnp.sum(j[:, None] >= gtile[None, :], axis=1) - 1
    meta = jnp.stack([ch_dtile, ch_stile, ch_first, ch_last, ch_valid])

    x_b = x.astype(jnp.bfloat16)
    w_b = weight.astype(jnp.bfloat16)
    b_p = bias.astype(jnp.float32)[None, :]

    y = pl.pallas_call(
        _xform_kernel,
        out_shape=jax.ShapeDtypeStruct((n, nhid), jnp.bfloat16),
        grid=(n // tm,),
        in_specs=[
            pl.BlockSpec((tm, nfeat), lambda i: (i, 0)),
            pl.BlockSpec((nfeat, nhid), lambda i: (0, 0)),
            pl.BlockSpec((tm, 1), lambda i: (i, 0)),
        ],
        out_specs=pl.BlockSpec((tm, nhid), lambda i: (i, 0)),
        compiler_params=pltpu.CompilerParams(
            dimension_semantics=("parallel",),
            vmem_limit_bytes=_VMEM_LIMIT),
    )(x_b, w_b, d_is)

    out = pl.pallas_call(
        _make_agg_kernel(nch, tm, ts, c),
        out_shape=jax.ShapeDtypeStruct((n, nhid), jnp.float32),
        grid_spec=pltpu.PrefetchScalarGridSpec(
            num_scalar_prefetch=1,
            grid=(2, nch // 2),
            in_specs=[
                pl.BlockSpec((nc, 1, c), lambda h, j, m: (0, 0, 0)),
                pl.BlockSpec((nc, 1, c), lambda h, j, m: (0, 0, 0)),
                pl.BlockSpec((n, nhid), lambda h, j, m: (0, 0)),
                pl.BlockSpec((n, 1), lambda h, j, m: (0, 0)),
                pl.BlockSpec((1, nhid), lambda h, j, m: (0, 0)),
            ],
            out_specs=pl.BlockSpec((tm, nhid),
                                   lambda h, j, m: (m[0, h * nch + 2 * j], 0)),
            scratch_shapes=[pltpu.VMEM((tm, nhid), jnp.float32)],
        ),
        compiler_params=pltpu.CompilerParams(
            dimension_semantics=("parallel", "arbitrary"),
            vmem_limit_bytes=_VMEM_LIMIT),
    )(meta, srcl3, dstl3, y, d_is, b_p)

    return out


# speedup vs baseline: 1.4310x; 1.4310x over previous
import jax
import jax.numpy as jnp
from jax import lax
from jax.experimental import pallas as pl
from jax.experimental.pallas import tpu as pltpu

_VMEM_LIMIT = 64 * 1024 * 1024


def _xform_kernel(x_ref, w_ref, dis_ref, y_ref):
    xw = jnp.dot(x_ref[...], w_ref[...], preferred_element_type=jnp.float32)
    y_ref[...] = (dis_ref[...] * xw).astype(y_ref.dtype)


def _make_agg_kernel(nch, tm, ts, c):
    def _contrib(srcl_ref, dstl_ref, y_ref, jg, st):
        srcl = srcl_ref[jg, 0, :]
        dstl = dstl_ref[jg, 0, :]
        ys = y_ref[pl.ds(pl.multiple_of(st * ts, ts), ts), :]
        s_oht = (lax.broadcasted_iota(jnp.int32, (ts, c), 0)
                 == srcl[None, :]).astype(jnp.bfloat16)
        g = lax.dot_general(s_oht, ys, (((0,), (0,)), ((), ())),
                            preferred_element_type=jnp.float32)
        d_oh = (lax.broadcasted_iota(jnp.int32, (tm, c), 0)
                == dstl[None, :]).astype(jnp.bfloat16)
        return jnp.dot(d_oh, g.astype(jnp.bfloat16),
                       preferred_element_type=jnp.float32)

    def _agg_kernel(meta_ref, srcl_ref, dstl_ref, y_ref, dis_ref, b_ref,
                    o_ref, acc_ref):
        h = pl.program_id(0)
        j = pl.program_id(1)
        jg = h * nch + 4 * j
        dt = meta_ref[0, jg]

        @pl.when(meta_ref[2, jg] == 1)
        def _():
            acc_ref[...] = jnp.zeros_like(acc_ref)

        @pl.when(meta_ref[4, jg] == 1)
        def _():
            cs = [_contrib(srcl_ref, dstl_ref, y_ref, jg + k,
                           meta_ref[1, jg + k]) for k in range(4)]
            acc_ref[...] += (cs[0] + cs[1]) + (cs[2] + cs[3])

        @pl.when(meta_ref[3, jg + 3] == 1)
        def _():
            yd = y_ref[pl.ds(pl.multiple_of(dt * tm, tm), tm), :]
            dis = dis_ref[pl.ds(pl.multiple_of(dt * tm, tm), tm), :]
            o_ref[...] = (dis * (acc_ref[...] + yd.astype(jnp.float32))
                          + b_ref[...])

    return _agg_kernel


def kernel(x, edge_index, weight, bias):
    n, nfeat = x.shape
    nhid = weight.shape[1]
    e = edge_index.shape[1]
    i32 = jnp.int32

    tm = 512
    ts = 512
    c = 256
    nt = n // tm
    ns = n // ts
    nth = nt // 2
    npair = nt * ns
    nch = e // c + 1 + nth * ns + 7 * nth + 3
    nch += (-nch) % 4
    nc = 2 * nch

    src = edge_index[0]
    dst = edge_index[1]

    deg = jnp.zeros((n,), jnp.float32).at[dst].add(1.0) + 1.0
    d_is = lax.rsqrt(deg)[:, None]

    sb = (n - 1).bit_length()
    db = (tm - 1).bit_length()
    dtile = dst // tm
    stile = src // ts
    key = (dtile * ns + stile).astype(i32)
    kv = jnp.sort((key << (sb + db)) | (src << db) | (dst % tm))
    skey = kv >> (sb + db)
    src_s = (kv >> db) & (n - 1)
    srcl_s = src_s % ts
    stile_s = src_s // ts
    dstl_s = kv & (tm - 1)

    cnt = jnp.zeros((npair,), i32).at[key].add(1)
    ch_p = (cnt + c - 1) // c
    tile_ch = ch_p.reshape(nt, ns).sum(1)
    tile_ch_m = jnp.maximum(tile_ch, 1)
    tile_ch_m = tile_ch_m + ((-tile_ch_m) % 4)
    d_t = jnp.cumsum(tile_ch) - tile_ch
    g_ex = jnp.cumsum(tile_ch_m) - tile_ch_m
    half = (jnp.arange(nt, dtype=i32) >= nth).astype(i32)
    gtile = half * nch + g_ex - half * g_ex[nth]

    idx = jnp.arange(e, dtype=i32)
    chg = jnp.concatenate([jnp.ones((1,), i32),
                           (skey[1:] != skey[:-1]).astype(i32)])
    run_start = lax.cummax(idx * chg)
    r = idx - run_start
    c_ord = jnp.cumsum((r % c == 0).astype(i32)) - 1
    adj = gtile - d_t
    dtile_e = skey // ns
    oh16 = (dtile_e[:, None] == jnp.arange(nt, dtype=i32)[None, :])
    slot = c_ord + jnp.sum(oh16 * adj[None, :], axis=1)
    pos = slot * c + r % c

    pack = (stile_s << 20) | (srcl_s << 10) | dstl_s
    sentp = (1023 << 10) | 1023
    buf = jnp.full((nc * c,), sentp, i32).at[pos].add(pack - sentp)
    buf2 = buf.reshape(nc, c)
    srcl3 = ((buf2 >> 10) & 1023).reshape(nc, 1, c)
    dstl3 = (buf2 & 1023).reshape(nc, 1, c)
    head = buf2[:, 0]
    ch_stile = head >> 20
    ch_valid = (head != sentp).astype(i32)

    j = jnp.arange(nc, dtype=i32)
    ch_first = jnp.any(j[:, None] == gtile[None, :], axis=1).astype(i32)
    ch_last = jnp.any(j[:, None] == (gtile + tile_ch_m - 1)[None, :],
                      axis=1).astype(i32)
    ch_dtile = jnp.sum(j[:, None] >= gtile[None, :], axis=1) - 1
    meta = jnp.stack([ch_dtile, ch_stile, ch_first, ch_last, ch_valid])

    x_b = x.astype(jnp.bfloat16)
    w_b = weight.astype(jnp.bfloat16)
    b_p = bias.astype(jnp.float32)[None, :]

    y = pl.pallas_call(
        _xform_kernel,
        out_shape=jax.ShapeDtypeStruct((n, nhid), jnp.bfloat16),
        grid=(n // tm,),
        in_specs=[
            pl.BlockSpec((tm, nfeat), lambda i: (i, 0)),
            pl.BlockSpec((nfeat, nhid), lambda i: (0, 0)),
            pl.BlockSpec((tm, 1), lambda i: (i, 0)),
        ],
        out_specs=pl.BlockSpec((tm, nhid), lambda i: (i, 0)),
        compiler_params=pltpu.CompilerParams(
            dimension_semantics=("parallel",),
            vmem_limit_bytes=_VMEM_LIMIT),
    )(x_b, w_b, d_is)

    out = pl.pallas_call(
        _make_agg_kernel(nch, tm, ts, c),
        out_shape=jax.ShapeDtypeStruct((n, nhid), jnp.float32),
        grid_spec=pltpu.PrefetchScalarGridSpec(
            num_scalar_prefetch=1,
            grid=(2, nch // 4),
            in_specs=[
                pl.BlockSpec((nc, 1, c), lambda h, j, m: (0, 0, 0)),
                pl.BlockSpec((nc, 1, c), lambda h, j, m: (0, 0, 0)),
                pl.BlockSpec((n, nhid), lambda h, j, m: (0, 0)),
                pl.BlockSpec((n, 1), lambda h, j, m: (0, 0)),
                pl.BlockSpec((1, nhid), lambda h, j, m: (0, 0)),
            ],
            out_specs=pl.BlockSpec((tm, nhid),
                                   lambda h, j, m: (m[0, h * nch + 4 * j], 0)),
            scratch_shapes=[pltpu.VMEM((tm, nhid), jnp.float32)],
        ),
        compiler_params=pltpu.CompilerParams(
            dimension_semantics=("parallel", "arbitrary"),
            vmem_limit_bytes=_VMEM_LIMIT),
    )(meta, srcl3, dstl3, y, d_is, b_p)

    return out


# speedup vs baseline: 1.4855x; 1.0381x over previous
import jax
import jax.numpy as jnp
from jax import lax
from jax.experimental import pallas as pl
from jax.experimental.pallas import tpu as pltpu

_VMEM_LIMIT = 64 * 1024 * 1024


def _xform_kernel(x_ref, w_ref, dis_ref, y_ref):
    xw = jnp.dot(x_ref[...], w_ref[...], preferred_element_type=jnp.float32)
    y_ref[...] = (dis_ref[...] * xw).astype(y_ref.dtype)


def _make_agg_kernel(nch, tm, ts, c):
    def _contrib(srcl_ref, dstl_ref, y_ref, jg, st):
        srcl = srcl_ref[jg, 0, :]
        dstl = dstl_ref[jg, 0, :]
        ys = y_ref[pl.ds(pl.multiple_of(st * ts, ts), ts), :]
        s_oht = (lax.broadcasted_iota(jnp.int32, (ts, c), 0)
                 == srcl[None, :]).astype(jnp.bfloat16)
        g = lax.dot_general(s_oht, ys, (((0,), (0,)), ((), ())),
                            preferred_element_type=jnp.float32)
        d_oh = (lax.broadcasted_iota(jnp.int32, (tm, c), 0)
                == dstl[None, :]).astype(jnp.bfloat16)
        return jnp.dot(d_oh, g.astype(jnp.bfloat16),
                       preferred_element_type=jnp.float32)

    def _agg_kernel(meta_ref, srcl_ref, dstl_ref, y_ref, dis_ref, b_ref,
                    o_ref, acc_ref):
        h = pl.program_id(0)
        j = pl.program_id(1)
        jg = h * nch + 8 * j
        dt = meta_ref[0, jg]

        @pl.when(meta_ref[2, jg] == 1)
        def _():
            acc_ref[...] = jnp.zeros_like(acc_ref)

        @pl.when(meta_ref[4, jg] == 1)
        def _():
            cs = [_contrib(srcl_ref, dstl_ref, y_ref, jg + k,
                           meta_ref[1, jg + k]) for k in range(8)]
            acc_ref[...] += (((cs[0] + cs[1]) + (cs[2] + cs[3]))
                             + ((cs[4] + cs[5]) + (cs[6] + cs[7])))

        @pl.when(meta_ref[3, jg + 7] == 1)
        def _():
            yd = y_ref[pl.ds(pl.multiple_of(dt * tm, tm), tm), :]
            dis = dis_ref[pl.ds(pl.multiple_of(dt * tm, tm), tm), :]
            o_ref[...] = (dis * (acc_ref[...] + yd.astype(jnp.float32))
                          + b_ref[...])

    return _agg_kernel


def kernel(x, edge_index, weight, bias):
    n, nfeat = x.shape
    nhid = weight.shape[1]
    e = edge_index.shape[1]
    i32 = jnp.int32

    tm = 512
    ts = 512
    c = 256
    nt = n // tm
    ns = n // ts
    nth = nt // 2
    npair = nt * ns
    nch = e // c + 1 + nth * ns + 15 * nth + 7
    nch += (-nch) % 8
    nc = 2 * nch

    src = edge_index[0]
    dst = edge_index[1]

    deg = jnp.zeros((n,), jnp.float32).at[dst].add(1.0) + 1.0
    d_is = lax.rsqrt(deg)[:, None]

    sb = (n - 1).bit_length()
    db = (tm - 1).bit_length()
    dtile = dst // tm
    stile = src // ts
    key = (dtile * ns + stile).astype(i32)
    kv = jnp.sort((key << (sb + db)) | (src << db) | (dst % tm))
    skey = kv >> (sb + db)
    src_s = (kv >> db) & (n - 1)
    srcl_s = src_s % ts
    stile_s = src_s // ts
    dstl_s = kv & (tm - 1)

    cnt = jnp.zeros((npair,), i32).at[key].add(1)
    ch_p = (cnt + c - 1) // c
    tile_ch = ch_p.reshape(nt, ns).sum(1)
    tile_ch_m = jnp.maximum(tile_ch, 1)
    tile_ch_m = tile_ch_m + ((-tile_ch_m) % 8)
    d_t = jnp.cumsum(tile_ch) - tile_ch
    g_ex = jnp.cumsum(tile_ch_m) - tile_ch_m
    half = (jnp.arange(nt, dtype=i32) >= nth).astype(i32)
    gtile = half * nch + g_ex - half * g_ex[nth]

    idx = jnp.arange(e, dtype=i32)
    chg = jnp.concatenate([jnp.ones((1,), i32),
                           (skey[1:] != skey[:-1]).astype(i32)])
    run_start = lax.cummax(idx * chg)
    r = idx - run_start
    c_ord = jnp.cumsum((r % c == 0).astype(i32)) - 1
    adj = gtile - d_t
    dtile_e = skey // ns
    oh16 = (dtile_e[:, None] == jnp.arange(nt, dtype=i32)[None, :])
    slot = c_ord + jnp.sum(oh16 * adj[None, :], axis=1)
    pos = slot * c + r % c

    pack = (stile_s << 20) | (srcl_s << 10) | dstl_s
    sentp = (1023 << 10) | 1023
    buf = jnp.full((nc * c,), sentp, i32).at[pos].add(pack - sentp)
    buf2 = buf.reshape(nc, c)
    srcl3 = ((buf2 >> 10) & 1023).reshape(nc, 1, c)
    dstl3 = (buf2 & 1023).reshape(nc, 1, c)
    head = buf2[:, 0]
    ch_stile = head >> 20
    ch_valid = (head != sentp).astype(i32)

    j = jnp.arange(nc, dtype=i32)
    ch_first = jnp.any(j[:, None] == gtile[None, :], axis=1).astype(i32)
    ch_last = jnp.any(j[:, None] == (gtile + tile_ch_m - 1)[None, :],
                      axis=1).astype(i32)
    ch_dtile = jnp.sum(j[:, None] >= gtile[None, :], axis=1) - 1
    meta = jnp.stack([ch_dtile, ch_stile, ch_first, ch_last, ch_valid])

    x_b = x.astype(jnp.bfloat16)
    w_b = weight.astype(jnp.bfloat16)
    b_p = bias.astype(jnp.float32)[None, :]

    y = pl.pallas_call(
        _xform_kernel,
        out_shape=jax.ShapeDtypeStruct((n, nhid), jnp.bfloat16),
        grid=(n // tm,),
        in_specs=[
            pl.BlockSpec((tm, nfeat), lambda i: (i, 0)),
            pl.BlockSpec((nfeat, nhid), lambda i: (0, 0)),
            pl.BlockSpec((tm, 1), lambda i: (i, 0)),
        ],
        out_specs=pl.BlockSpec((tm, nhid), lambda i: (i, 0)),
        compiler_params=pltpu.CompilerParams(
            dimension_semantics=("parallel",),
            vmem_limit_bytes=_VMEM_LIMIT),
    )(x_b, w_b, d_is)

    out = pl.pallas_call(
        _make_agg_kernel(nch, tm, ts, c),
        out_shape=jax.ShapeDtypeStruct((n, nhid), jnp.float32),
        grid_spec=pltpu.PrefetchScalarGridSpec(
            num_scalar_prefetch=1,
            grid=(2, nch // 8),
            in_specs=[
                pl.BlockSpec((nc, 1, c), lambda h, j, m: (0, 0, 0)),
                pl.BlockSpec((nc, 1, c), lambda h, j, m: (0, 0, 0)),
                pl.BlockSpec((n, nhid), lambda h, j, m: (0, 0)),
                pl.BlockSpec((n, 1), lambda h, j, m: (0, 0)),
                pl.BlockSpec((1, nhid), lambda h, j, m: (0, 0)),
            ],
            out_specs=pl.BlockSpec((tm, nhid),
                                   lambda h, j, m: (m[0, h * nch + 8 * j], 0)),
            scratch_shapes=[pltpu.VMEM((tm, nhid), jnp.float32)],
        ),
        compiler_params=pltpu.CompilerParams(
            dimension_semantics=("parallel", "arbitrary"),
            vmem_limit_bytes=_VMEM_LIMIT),
    )(meta, srcl3, dstl3, y, d_is, b_p)

    return out
